# baseline (speedup 1.0000x reference)
import jax
import jax.numpy as jnp
from jax.experimental import pallas as pl

_BLOCK_ROWS = 1024


def _pad_body(x_ref, out_ref, mask_ref):
    out_ref[...] = x_ref[...]

    @pl.when(pl.program_id(0) == 0)
    def _():
        mask_ref[...] = jnp.ones(mask_ref.shape, dtype=jnp.bool_)


def kernel(bags):
    b, s, d = bags.shape
    n = b * s
    flat = bags.reshape(n, d)
    rows = min(_BLOCK_ROWS, n)
    padded, mask = pl.pallas_call(
        _pad_body,
        grid=(pl.cdiv(n, rows),),
        in_specs=[pl.BlockSpec((rows, d), lambda i: (i, 0))],
        out_specs=(
            pl.BlockSpec((rows, d), lambda i: (i, 0)),
            pl.BlockSpec((b, s), lambda i: (0, 0)),
        ),
        out_shape=(
            jax.ShapeDtypeStruct((n, d), bags.dtype),
            jax.ShapeDtypeStruct((b, s), jnp.bool_),
        ),
    )(flat)
    return (padded.reshape(b, s, d), mask)


# speedup vs baseline: 1.1064x; 1.1064x over previous
import jax
import jax.numpy as jnp
from jax.experimental import pallas as pl

_BLOCK_ROWS = 4096


def _pad_body(x_ref, out_ref, mask_ref):
    out_ref[...] = x_ref[...]

    @pl.when(pl.program_id(0) == 0)
    def _():
        mask_ref[...] = jnp.ones(mask_ref.shape, dtype=jnp.bool_)


def kernel(bags):
    b, s, d = bags.shape
    n = b * s
    flat = bags.reshape(n, d)
    rows = min(_BLOCK_ROWS, n)
    padded, mask = pl.pallas_call(
        _pad_body,
        grid=(pl.cdiv(n, rows),),
        in_specs=[pl.BlockSpec((rows, d), lambda i: (i, 0))],
        out_specs=(
            pl.BlockSpec((rows, d), lambda i: (i, 0)),
            pl.BlockSpec((b, s), lambda i: (0, 0)),
        ),
        out_shape=(
            jax.ShapeDtypeStruct((n, d), bags.dtype),
            jax.ShapeDtypeStruct((b, s), jnp.bool_),
        ),
    )(flat)
    return (padded.reshape(b, s, d), mask)
